# Initial kernel scaffold; baseline (speedup 1.0000x reference)
import jax
import jax.numpy as jnp
from jax.experimental import pallas as pl


def kernel(item_ids, frozen_emb, item_table, ln_gamma, ln_beta):
    raise NotImplementedError("write your pallas kernel here")


# speedup vs baseline: 1.8851x; 1.8851x over previous
import functools

import jax
import jax.numpy as jnp
from jax import lax
from jax.experimental import pallas as pl
from jax.experimental.pallas import tpu as pltpu
from jax.experimental.pallas import tpu_sc as plsc

_LN_EPS = 1e-5


def _ln_body(x_ref, g_ref, b_ref, o_ref):
    x = x_ref[...]
    mean = jnp.mean(x, axis=-1, keepdims=True)
    var = jnp.mean((x - mean) ** 2, axis=-1, keepdims=True)
    y = (x - mean) / jnp.sqrt(var + _LN_EPS) * g_ref[...] + b_ref[...]
    o_ref[...] = jnp.concatenate([y, jnp.zeros_like(y)], axis=-1)


def _ln_table(table, gamma, beta):
    v, d = table.shape
    rows = 2000
    grid = v // rows
    return pl.pallas_call(
        _ln_body,
        grid=(grid,),
        in_specs=[
            pl.BlockSpec((rows, d), lambda i: (i, 0)),
            pl.BlockSpec((1, d), lambda i: (0, 0)),
            pl.BlockSpec((1, d), lambda i: (0, 0)),
        ],
        out_specs=pl.BlockSpec((rows, 2 * d), lambda i: (i, 0)),
        out_shape=jax.ShapeDtypeStruct((v, 2 * d), jnp.float32),
    )(table, gamma.reshape(1, d), beta.reshape(1, d))


def _make_sc_gather(n, v, df, dn, chunk):
    info = plsc.get_sparse_core_info()
    nw = info.num_cores * info.num_subcores
    n_per_w = n // nw
    steps = n_per_w // chunk
    mesh = plsc.VectorSubcoreMesh(core_axis_name="c", subcore_axis_name="s")

    @functools.partial(
        pl.kernel,
        out_type=jax.ShapeDtypeStruct((n, df + dn), jnp.float32),
        mesh=mesh,
        scratch_types=[
            pltpu.VMEM((chunk,), jnp.int32),
            pltpu.VMEM((chunk, df + dn), jnp.float32),
            pltpu.VMEM((chunk, 2 * dn), jnp.float32),
            pltpu.SemaphoreType.DMA,
            pltpu.SemaphoreType.DMA,
        ],
    )
    def sc_gather(idx_hbm, frozen_hbm, normed_hbm, out_hbm,
                  idx_v, stage_v, norm_v, sem_f, sem_n):
        wid = lax.axis_index("s") * info.num_cores + lax.axis_index("c")
        w_base = wid * n_per_w
        lanes = info.num_lanes

        def body(g, carry):
            base = w_base + g * chunk
            pltpu.sync_copy(idx_hbm.at[pl.ds(base, chunk)], idx_v)
            cp_f = pltpu.async_copy(frozen_hbm.at[idx_v],
                                    stage_v.at[:, pl.ds(0, df)], sem_f)
            cp_n = pltpu.async_copy(normed_hbm.at[idx_v], norm_v, sem_n)
            cp_f.wait()
            cp_n.wait()
            for r in range(chunk):
                for j in range(dn // lanes):
                    stage_v[r, pl.ds(df + j * lanes, lanes)] = (
                        norm_v[r, pl.ds(j * lanes, lanes)])
            pltpu.sync_copy(stage_v, out_hbm.at[pl.ds(base, chunk)])
            return carry

        lax.fori_loop(0, steps, body, 0, unroll=False)

    return sc_gather


def kernel(item_ids, frozen_emb, item_table, ln_gamma, ln_beta):
    b, l = item_ids.shape
    v, df = frozen_emb.shape
    dn = item_table.shape[1]
    n = b * l

    normed = _ln_table(item_table, ln_gamma, ln_beta)
    idx = item_ids.reshape(n).astype(jnp.int32)
    out = _make_sc_gather(n, v, df, dn, chunk=64)(idx, frozen_emb, normed)
    return out.reshape(b, l, df + dn)


# speedup vs baseline: 2.1250x; 1.1273x over previous
import functools

import jax
import jax.numpy as jnp
from jax import lax
from jax.experimental import pallas as pl
from jax.experimental.pallas import tpu as pltpu
from jax.experimental.pallas import tpu_sc as plsc

_LN_EPS = 1e-5


def _ln_body(x_ref, g_ref, b_ref, o_ref):
    x = x_ref[...]
    mean = jnp.mean(x, axis=-1, keepdims=True)
    var = jnp.mean((x - mean) ** 2, axis=-1, keepdims=True)
    y = (x - mean) / jnp.sqrt(var + _LN_EPS) * g_ref[...] + b_ref[...]
    o_ref[...] = jnp.concatenate([y, jnp.zeros_like(y)], axis=-1)


def _ln_table(table, gamma, beta):
    v, d = table.shape
    rows = 2000
    grid = v // rows
    return pl.pallas_call(
        _ln_body,
        grid=(grid,),
        in_specs=[
            pl.BlockSpec((rows, d), lambda i: (i, 0)),
            pl.BlockSpec((1, d), lambda i: (0, 0)),
            pl.BlockSpec((1, d), lambda i: (0, 0)),
        ],
        out_specs=pl.BlockSpec((rows, 2 * d), lambda i: (i, 0)),
        out_shape=jax.ShapeDtypeStruct((v, 2 * d), jnp.float32),
    )(table, gamma.reshape(1, d), beta.reshape(1, d))


def _make_sc_gather(n, v, df, dn, chunk, nbuf=2):
    info = plsc.get_sparse_core_info()
    nw = info.num_cores * info.num_subcores
    n_per_w = n // nw
    steps = n_per_w // chunk
    mesh = plsc.VectorSubcoreMesh(core_axis_name="c", subcore_axis_name="s")

    @functools.partial(
        pl.kernel,
        out_type=jax.ShapeDtypeStruct((n, df + dn), jnp.float32),
        mesh=mesh,
        scratch_types=[
            pltpu.VMEM((n_per_w,), jnp.int32),
        ] + [pltpu.VMEM((chunk, df + dn), jnp.float32)] * nbuf
          + [pltpu.VMEM((chunk, 2 * dn), jnp.float32)] * nbuf
          + [pltpu.SemaphoreType.DMA] * (3 * nbuf),
    )
    def sc_gather(idx_hbm, frozen_hbm, normed_hbm, out_hbm, idx_all, *bufs):
        stage = bufs[:nbuf]
        norm = bufs[nbuf:2 * nbuf]
        sem_f = bufs[2 * nbuf:3 * nbuf]
        sem_n = bufs[3 * nbuf:4 * nbuf]
        sem_w = bufs[4 * nbuf:5 * nbuf]
        wid = lax.axis_index("s") * info.num_cores + lax.axis_index("c")
        w_base = wid * n_per_w
        lanes = info.num_lanes

        pltpu.sync_copy(idx_hbm.at[pl.ds(w_base, n_per_w)], idx_all)

        def gather(g, s):
            rows = idx_all.at[pl.ds(g * chunk, chunk)]
            cf = pltpu.make_async_copy(frozen_hbm.at[rows],
                                       stage[s].at[:, pl.ds(0, df)], sem_f[s])
            cn = pltpu.make_async_copy(normed_hbm.at[rows], norm[s], sem_n[s])
            return cf, cn

        def write(g, s):
            return pltpu.make_async_copy(
                stage[s], out_hbm.at[pl.ds(w_base + g * chunk, chunk)],
                sem_w[s])

        for s in range(nbuf):
            cf, cn = gather(s, s)
            cf.start()
            cn.start()

        def body(big, carry):
            for s in range(nbuf):
                g = big * nbuf + s
                cf, cn = gather(g, s)
                cf.wait()
                cn.wait()
                for r in range(chunk):
                    for j in range(dn // lanes):
                        stage[s][r, pl.ds(df + j * lanes, lanes)] = (
                            norm[s][r, pl.ds(j * lanes, lanes)])
                write(g, s).start()
                nxt = g + nbuf

                @pl.when(nxt < steps)
                def _():
                    write(g, s).wait()
                    nf, nn = gather(nxt, s)
                    nf.start()
                    nn.start()
            return carry

        lax.fori_loop(0, steps // nbuf, body, 0, unroll=False)
        for s in range(nbuf):
            write(steps - nbuf + s, s).wait()

    return sc_gather


def kernel(item_ids, frozen_emb, item_table, ln_gamma, ln_beta):
    b, l = item_ids.shape
    v, df = frozen_emb.shape
    dn = item_table.shape[1]
    n = b * l

    normed = _ln_table(item_table, ln_gamma, ln_beta)
    idx = item_ids.reshape(n).astype(jnp.int32)
    out = _make_sc_gather(n, v, df, dn, chunk=64)(idx, frozen_emb, normed)
    return out.reshape(b, l, df + dn)
